# baseline (speedup 1.0000x reference)
import functools

import jax
import jax.numpy as jnp
import numpy as np
from jax import lax
from jax.experimental import pallas as pl
from jax.experimental.pallas import tpu as pltpu
from jax.experimental.pallas import tpu_sc as plsc

NW = 32
CAP = 128
E = NW * CAP
N_NODES = 10000
N_EDGES = 160000
CHUNK = 5000
TBL = 10248
H = 128
NH = 4
HD = H // NH


def _sc_body(ei_hbm, ts_hbm, tgt_hbm, nf_hbm, zeros_hbm,
             tgtid_out, nbr_out, ts_out, g_out, t_out,
             tbl, e0c, e1c, tsc, tgtv, tgtbuf, nbrbuf, tsbuf, rows, trows,
             sem, sem2):
    wid = lax.axis_index("s") * 2 + lax.axis_index("c")
    base = wid * CHUNK
    c0 = pltpu.async_copy(ei_hbm.at[pl.ds(base, CHUNK)],
                          e0c.at[pl.ds(0, CHUNK)], sem)
    c1 = pltpu.async_copy(ei_hbm.at[pl.ds(N_EDGES + base, CHUNK)],
                          e1c.at[pl.ds(0, CHUNK)], sem)
    c2 = pltpu.async_copy(ts_hbm.at[pl.ds(base, CHUNK)],
                          tsc.at[pl.ds(0, CHUNK)], sem)
    c3 = pltpu.async_copy(tgt_hbm, tgtv, sem)
    c4 = pltpu.async_copy(zeros_hbm, tbl, sem2)

    zeros_f = jnp.zeros((16,), jnp.float32)
    neg_i = jnp.full((16,), -1, jnp.int32)
    ones_i = jnp.ones((16,), jnp.int32)
    lane = lax.iota(jnp.int32, 16)

    for j in range(CAP // 16):
        tgtbuf[pl.ds(j * 16, 16)] = neg_i
        nbrbuf[pl.ds(j * 16, 16)] = lane * 16 + j
        tsbuf[pl.ds(j * 16, 16)] = zeros_f

    c0.wait()
    c1.wait()
    c2.wait()
    c3.wait()
    c4.wait()

    for j in range(64 // 16):
        idx = tgtv[pl.ds(j * 16, 16)]
        plsc.store_scatter(tbl, [idx], ones_i)

    def append16(e0, e1, tv, c):
        f0 = plsc.load_gather(tbl, [e0])
        f1 = plsc.load_gather(tbl, [e1])
        m0 = f0 > 0
        m1 = (f1 > 0) & (e0 != e1)
        anym = jnp.any(m0 | m1)

        def app(c):
            b0 = jnp.minimum(c, CAP - 16)
            plsc.store_compressed(tgtbuf.at[pl.ds(b0, 16)], e0, mask=m0)
            plsc.store_compressed(nbrbuf.at[pl.ds(b0, 16)], e1, mask=m0)
            plsc.store_compressed(tsbuf.at[pl.ds(b0, 16)], tv, mask=m0)
            c = c + jnp.sum(m0.astype(jnp.int32))
            b1 = jnp.minimum(c, CAP - 16)
            plsc.store_compressed(tgtbuf.at[pl.ds(b1, 16)], e1, mask=m1)
            plsc.store_compressed(nbrbuf.at[pl.ds(b1, 16)], e0, mask=m1)
            plsc.store_compressed(tsbuf.at[pl.ds(b1, 16)], tv, mask=m1)
            return c + jnp.sum(m1.astype(jnp.int32))

        return lax.cond(anym, app, lambda c: c, c)

    def body(i, cnt):
        e0 = e0c[pl.ds(i * 16, 16)]
        e1 = e1c[pl.ds(i * 16, 16)]
        tv = tsc[pl.ds(i * 16, 16)]
        return append16(e0, e1, tv, cnt)

    cnt = lax.fori_loop(0, CHUNK // 16, body, jnp.int32(0))

    tail_ok = lane < (CHUNK % 16)
    e0t = jnp.where(tail_ok, e0c[pl.ds(CHUNK - 8, 16)], N_NODES)
    e1t = jnp.where(tail_ok, e1c[pl.ds(CHUNK - 8, 16)], N_NODES)
    tvt = jnp.where(tail_ok, tsc[pl.ds(CHUNK - 8, 16)], 0.0)
    append16(e0t, e1t, tvt, cnt)

    pltpu.async_copy(nf_hbm.at[nbrbuf], rows, sem).wait()

    pltpu.sync_copy(tgtbuf, tgtid_out.at[pl.ds(wid * CAP, CAP)])
    pltpu.sync_copy(nbrbuf, nbr_out.at[pl.ds(wid * CAP, CAP)])
    pltpu.sync_copy(tsbuf, ts_out.at[pl.ds(wid * CAP, CAP)])
    pltpu.sync_copy(rows, g_out.at[pl.ds(wid * CAP, CAP)])

    @pl.when(wid == 0)
    def _():
        pltpu.async_copy(nf_hbm.at[tgtv], trows, sem).wait()
        pltpu.sync_copy(trows, t_out)


def _sc_compact(ei, ts, tgt_ids, node_features, interpret=False):
    f32, i32 = jnp.float32, jnp.int32
    return pl.kernel(
        _sc_body,
        out_type=[
            jax.ShapeDtypeStruct((E,), i32),
            jax.ShapeDtypeStruct((E,), i32),
            jax.ShapeDtypeStruct((E,), f32),
            jax.ShapeDtypeStruct((E, H), f32),
            jax.ShapeDtypeStruct((64, H), f32),
        ],
        mesh=plsc.VectorSubcoreMesh(core_axis_name="c", subcore_axis_name="s"),
        scratch_types=[
            pltpu.VMEM((TBL,), i32),
            pltpu.VMEM((CHUNK + 8,), i32),
            pltpu.VMEM((CHUNK + 8,), i32),
            pltpu.VMEM((CHUNK + 8,), f32),
            pltpu.VMEM((64,), i32),
            pltpu.VMEM((CAP,), i32),
            pltpu.VMEM((CAP,), i32),
            pltpu.VMEM((CAP,), f32),
            pltpu.VMEM((CAP, H), f32),
            pltpu.VMEM((64, H), f32),
            pltpu.SemaphoreType.DMA,
            pltpu.SemaphoreType.DMA,
        ],
        compiler_params=pltpu.CompilerParams(needs_layout_passes=False),
        interpret=interpret,
    )(ei, ts, tgt_ids, node_features, jnp.zeros((TBL,), jnp.int32))


def _tc_body(*refs):
    (g_ref, t_ref, tgtid_ref, nbr_ref, ts_ref, tgtrow_ref,
     w_in_ref, b_in_ref) = refs[:8]
    layer_refs = refs[8:8 + 32]
    (w1_ref, b1_ref, w2_ref, b2_ref, wp1_ref, bp1_ref, wp2_ref, bp2_ref,
     wp3_ref, bp3_ref, out_ref) = refs[8 + 32:]

    f32 = jnp.float32

    def mm(a, b):
        return jnp.dot(a, b, preferred_element_type=f32)

    tgtid = tgtid_ref[...]
    nbr = nbr_ref[...]
    ts = ts_ref[...]
    tgtrow = tgtrow_ref[...]
    onehot = (tgtid == tgtrow).astype(f32)
    nbrhot = ((nbr == tgtrow) & (tgtid >= 0)).astype(f32)
    validf = (tgtid >= 0).astype(f32)

    r = lax.broadcasted_iota(jnp.int32, (H, NH), 0)
    c = lax.broadcasted_iota(jnp.int32, (H, NH), 1)
    hm = (r // HD == c).astype(f32)
    rt = lax.broadcasted_iota(jnp.int32, (NH, H), 0)
    ct = lax.broadcasted_iota(jnp.int32, (NH, H), 1)
    hmt = (ct // HD == rt).astype(f32)

    x_g = mm(g_ref[...], w_in_ref[...]) + b_in_ref[...]
    x_t = mm(t_ref[...], w_in_ref[...]) + b_in_ref[...]

    rowsum = jnp.maximum(jnp.sum(onehot, axis=1, keepdims=True), 1.0)
    nrs = jnp.sum(nbrhot, axis=1, keepdims=True)
    nrs_c = jnp.maximum(nrs, 1.0)
    inv_sqrt_hd = f32(1.0 / np.sqrt(HD))

    for l in range(2):
        (wf, bf, wt1, bt1, wt2, bt2, wq, bq, wk, bk, wv_, bv, wout, bout,
         wo, bo) = layer_refs[l * 16:(l + 1) * 16]
        tf_g = mm(x_g, wf[...]) + bf[...]
        tf_t = mm(x_t, wf[...]) + bf[...]
        t1 = jnp.maximum(ts * wt1[...] + bt1[...], 0.0)
        tfeat = mm(t1, wt2[...]) + bt2[...]
        nf = tf_g + tfeat
        q = mm(tf_t, wq[...]) + bq[...]
        k = mm(nf, wk[...]) + bk[...]
        v = mm(nf, wv_[...]) + bv[...]
        qrow = mm(onehot, q) / rowsum
        s = mm(qrow * k, hm) * inv_sqrt_hd
        w = jnp.exp(s) * validf
        den = lax.dot_general(onehot, w, (((0,), (0,)), ((), ())),
                              preferred_element_type=f32)
        wv = mm(w, hmt) * v
        num = lax.dot_general(onehot, wv, (((0,), (0,)), ((), ())),
                              preferred_element_type=f32)
        den_rep = mm(den, hmt)
        att = num / jnp.where(den_rep > 0, den_rep, 1.0)
        o = mm(att, wout[...]) + bout[...]
        hasedge = den_rep[:, 0:1] > 0
        agg = jnp.where(hasedge, o, tf_t)
        x_t = jnp.maximum(mm(agg, wo[...]) + bo[...], 0.0)
        sub = mm(nbrhot, x_t) / nrs_c
        x_g = jnp.where(nrs > 0, sub, jnp.maximum(x_g, 0.0))

    emb = mm(jnp.maximum(mm(x_t, w1_ref[...]) + b1_ref[...], 0.0),
             w2_ref[...]) + b2_ref[...]
    re = lax.broadcasted_iota(jnp.int32, (32, 64), 0)
    ce = lax.broadcasted_iota(jnp.int32, (32, 64), 1)
    sel_e = (ce == 2 * re).astype(f32)
    sel_o = (ce == 2 * re + 1).astype(f32)
    pair = jnp.concatenate([mm(sel_e, emb), mm(sel_o, emb)], axis=1)
    h1 = jnp.maximum(mm(pair, wp1_ref[...]) + bp1_ref[...], 0.0)
    h2 = jnp.maximum(mm(h1, wp2_ref[...]) + bp2_ref[...], 0.0)
    sc = mm(h2, wp3_ref[...]) + bp3_ref[...]
    out_ref[...] = 1.0 / (1.0 + jnp.exp(-sc))


def _tc_dense(args, interpret=False):
    return pl.pallas_call(
        _tc_body,
        out_shape=jax.ShapeDtypeStruct((32, 1), jnp.float32),
        interpret=interpret,
    )(*args)


def kernel(node_features, edge_index, edge_timestamps, target_pairs, params):
    i32 = jnp.int32
    tgt_ids = target_pairs.reshape(-1).astype(i32)

    tgtid, nbrid, tsg, g_rows, t_rows = _sc_compact(
        edge_index.reshape(-1), edge_timestamps, tgt_ids, node_features)

    p = params
    args = [g_rows, t_rows,
            tgtid.reshape(E, 1), nbrid.reshape(E, 1), tsg.reshape(E, 1),
            tgt_ids.reshape(1, 64),
            p['W_in'].T, p['b_in'].reshape(1, H)]
    for lp in p['layers']:
        in_w, in_b = lp['in_w'], lp['in_b']
        args += [
            lp['Wf'].T, lp['bf'].reshape(1, H),
            lp['Wt1'][:, 0].reshape(1, H), lp['bt1'].reshape(1, H),
            lp['Wt2'].T, lp['bt2'].reshape(1, H),
            in_w[:H].T, in_b[:H].reshape(1, H),
            in_w[H:2 * H].T, in_b[H:2 * H].reshape(1, H),
            in_w[2 * H:].T, in_b[2 * H:].reshape(1, H),
            lp['out_w'].T, lp['out_b'].reshape(1, H),
            lp['Wo'].T, lp['bo'].reshape(1, H),
        ]
    args += [p['W1'].T, p['b1'].reshape(1, H),
             p['W2'].T, p['b2'].reshape(1, 64),
             p['Wp1'].T, p['bp1'].reshape(1, H),
             p['Wp2'].T, p['bp2'].reshape(1, 64),
             p['Wp3'].T, p['bp3'].reshape(1, 1)]
    return _tc_dense(args)


# speedup vs baseline: 1.0111x; 1.0111x over previous
import functools

import jax
import jax.numpy as jnp
import numpy as np
from jax import lax
from jax.experimental import pallas as pl
from jax.experimental.pallas import tpu as pltpu
from jax.experimental.pallas import tpu_sc as plsc

NW = 32
CAP = 128
E = NW * CAP
N_NODES = 10000
N_EDGES = 160000
CHUNK = 5000
TBL = 10248
H = 128
NH = 4
HD = H // NH


def _sc_body(ei_hbm, ts_hbm, tgt_hbm, nf_hbm, zeros_hbm,
             tgtid_out, nbr_out, ts_out, g_out, t_out,
             tbl, e0c, e1c, tsc, tgtv, tgtbuf, nbrbuf, tsbuf, rows, trows,
             sem, sem2):
    wid = lax.axis_index("s") * 2 + lax.axis_index("c")
    base = wid * CHUNK
    c0 = pltpu.async_copy(ei_hbm.at[pl.ds(base, CHUNK)],
                          e0c.at[pl.ds(0, CHUNK)], sem)
    c1 = pltpu.async_copy(ei_hbm.at[pl.ds(N_EDGES + base, CHUNK)],
                          e1c.at[pl.ds(0, CHUNK)], sem)
    c2 = pltpu.async_copy(ts_hbm.at[pl.ds(base, CHUNK)],
                          tsc.at[pl.ds(0, CHUNK)], sem)
    c3 = pltpu.async_copy(tgt_hbm, tgtv, sem)
    c4 = pltpu.async_copy(zeros_hbm, tbl, sem2)

    zeros_f = jnp.zeros((16,), jnp.float32)
    neg_i = jnp.full((16,), -1, jnp.int32)
    ones_i = jnp.ones((16,), jnp.int32)
    lane = lax.iota(jnp.int32, 16)

    for j in range(CAP // 16):
        tgtbuf[pl.ds(j * 16, 16)] = neg_i
        nbrbuf[pl.ds(j * 16, 16)] = lane * 16 + j
        tsbuf[pl.ds(j * 16, 16)] = zeros_f

    c0.wait()
    c1.wait()
    c2.wait()
    c3.wait()
    c4.wait()

    for j in range(64 // 16):
        idx = tgtv[pl.ds(j * 16, 16)]
        plsc.store_scatter(tbl, [idx], ones_i)

    def append16(e0, e1, tv, c):
        f0 = plsc.load_gather(tbl, [e0])
        f1 = plsc.load_gather(tbl, [e1])
        m0 = f0 > 0
        m1 = (f1 > 0) & (e0 != e1)
        anym = jnp.any(m0 | m1)

        def app(c):
            b0 = jnp.minimum(c, CAP - 16)
            plsc.store_compressed(tgtbuf.at[pl.ds(b0, 16)], e0, mask=m0)
            plsc.store_compressed(nbrbuf.at[pl.ds(b0, 16)], e1, mask=m0)
            plsc.store_compressed(tsbuf.at[pl.ds(b0, 16)], tv, mask=m0)
            c = c + jnp.sum(m0.astype(jnp.int32))
            b1 = jnp.minimum(c, CAP - 16)
            plsc.store_compressed(tgtbuf.at[pl.ds(b1, 16)], e1, mask=m1)
            plsc.store_compressed(nbrbuf.at[pl.ds(b1, 16)], e0, mask=m1)
            plsc.store_compressed(tsbuf.at[pl.ds(b1, 16)], tv, mask=m1)
            return c + jnp.sum(m1.astype(jnp.int32))

        return lax.cond(anym, app, lambda c: c, c)

    def body(i, cnt):
        e0 = e0c[pl.ds(i * 16, 16)]
        e1 = e1c[pl.ds(i * 16, 16)]
        tv = tsc[pl.ds(i * 16, 16)]
        return append16(e0, e1, tv, cnt)

    cnt = lax.fori_loop(0, CHUNK // 16, body, jnp.int32(0))

    tail_ok = lane < (CHUNK % 16)
    e0t = jnp.where(tail_ok, e0c[pl.ds(CHUNK - 8, 16)], N_NODES)
    e1t = jnp.where(tail_ok, e1c[pl.ds(CHUNK - 8, 16)], N_NODES)
    tvt = jnp.where(tail_ok, tsc[pl.ds(CHUNK - 8, 16)], 0.0)
    append16(e0t, e1t, tvt, cnt)

    pltpu.async_copy(nf_hbm.at[nbrbuf], rows, sem).wait()

    pltpu.sync_copy(tgtbuf, tgtid_out.at[pl.ds(wid * CAP, CAP)])
    pltpu.sync_copy(nbrbuf, nbr_out.at[pl.ds(wid * CAP, CAP)])
    pltpu.sync_copy(tsbuf, ts_out.at[pl.ds(wid * CAP, CAP)])
    pltpu.sync_copy(rows, g_out.at[pl.ds(wid * CAP, CAP)])

    @pl.when(wid == 0)
    def _():
        pltpu.async_copy(nf_hbm.at[tgtv], trows, sem).wait()
        pltpu.sync_copy(trows, t_out)


def _sc_compact(ei, ts, tgt_ids, node_features, interpret=False):
    f32, i32 = jnp.float32, jnp.int32
    return pl.kernel(
        _sc_body,
        out_type=[
            jax.ShapeDtypeStruct((E,), i32),
            jax.ShapeDtypeStruct((E,), i32),
            jax.ShapeDtypeStruct((E,), f32),
            jax.ShapeDtypeStruct((E, H), f32),
            jax.ShapeDtypeStruct((64, H), f32),
        ],
        mesh=plsc.VectorSubcoreMesh(core_axis_name="c", subcore_axis_name="s"),
        scratch_types=[
            pltpu.VMEM((TBL,), i32),
            pltpu.VMEM((CHUNK + 8,), i32),
            pltpu.VMEM((CHUNK + 8,), i32),
            pltpu.VMEM((CHUNK + 8,), f32),
            pltpu.VMEM((64,), i32),
            pltpu.VMEM((CAP,), i32),
            pltpu.VMEM((CAP,), i32),
            pltpu.VMEM((CAP,), f32),
            pltpu.VMEM((CAP, H), f32),
            pltpu.VMEM((64, H), f32),
            pltpu.SemaphoreType.DMA,
            pltpu.SemaphoreType.DMA,
        ],
        compiler_params=pltpu.CompilerParams(needs_layout_passes=False),
        interpret=interpret,
    )(ei, ts, tgt_ids, node_features, jnp.zeros((TBL,), jnp.int32))


def _tc_body(*refs):
    (g_ref, t_ref, tgtid_ref, nbr_ref, ts_ref, tgtrow_ref,
     w_in_ref, b_in_ref) = refs[:8]
    layer_refs = refs[8:8 + 24]
    (w1_ref, b1_ref, w2_ref, b2_ref, wp1_ref, bp1_ref, wp2_ref, bp2_ref,
     wp3_ref, bp3_ref, out_ref) = refs[8 + 24:]

    f32 = jnp.float32

    def mmt(a, b):
        return lax.dot_general(a, b, (((1,), (1,)), ((), ())),
                               preferred_element_type=f32)

    def mm(a, b):
        return jnp.dot(a, b, preferred_element_type=f32)

    def lin(x, w_ref, b_ref):
        return mmt(x, w_ref[...]) + b_ref[...][None, :]

    tgtid = tgtid_ref[...]
    nbr = nbr_ref[...]
    ts = ts_ref[...]
    tgtrow = tgtrow_ref[...]
    onehot = (tgtid == tgtrow).astype(f32)
    nbrhot = ((nbr == tgtrow) & (tgtid >= 0)).astype(f32)
    validf = (tgtid >= 0).astype(f32)

    r = lax.broadcasted_iota(jnp.int32, (H, NH), 0)
    c = lax.broadcasted_iota(jnp.int32, (H, NH), 1)
    hm = (r // HD == c).astype(f32)
    rt = lax.broadcasted_iota(jnp.int32, (NH, H), 0)
    ct = lax.broadcasted_iota(jnp.int32, (NH, H), 1)
    hmt = (ct // HD == rt).astype(f32)

    x_g = lin(g_ref[...], w_in_ref, b_in_ref)
    x_t = lin(t_ref[...], w_in_ref, b_in_ref)

    rowsum = jnp.maximum(jnp.sum(onehot, axis=1, keepdims=True), 1.0)
    nrs = jnp.sum(nbrhot, axis=1, keepdims=True)
    nrs_c = jnp.maximum(nrs, 1.0)
    inv_sqrt_hd = f32(1.0 / np.sqrt(HD))

    for l in range(2):
        (wf, bf, wt1, bt1, wt2, bt2, in_w, in_b, wout, bout,
         wo, bo) = layer_refs[l * 12:(l + 1) * 12]
        tf_g = lin(x_g, wf, bf)
        tf_t = lin(x_t, wf, bf)
        t1 = jnp.maximum(mmt(ts, wt1[...]) + bt1[...][None, :], 0.0)
        tfeat = lin(t1, wt2, bt2)
        nf = tf_g + tfeat
        in_w_all = in_w[...]
        in_b_all = in_b[...]
        q = mmt(tf_t, in_w_all[:H]) + in_b_all[:H][None, :]
        k = mmt(nf, in_w_all[H:2 * H]) + in_b_all[H:2 * H][None, :]
        v = mmt(nf, in_w_all[2 * H:]) + in_b_all[2 * H:][None, :]
        qrow = mm(onehot, q) / rowsum
        s = mm(qrow * k, hm) * inv_sqrt_hd
        w = jnp.exp(s) * validf
        den = lax.dot_general(onehot, w, (((0,), (0,)), ((), ())),
                              preferred_element_type=f32)
        wv = mm(w, hmt) * v
        num = lax.dot_general(onehot, wv, (((0,), (0,)), ((), ())),
                              preferred_element_type=f32)
        den_rep = mm(den, hmt)
        att = num / jnp.where(den_rep > 0, den_rep, 1.0)
        o = lin(att, wout, bout)
        hasedge = den_rep[:, 0:1] > 0
        agg = jnp.where(hasedge, o, tf_t)
        x_t = jnp.maximum(lin(agg, wo, bo), 0.0)
        sub = mm(nbrhot, x_t) / nrs_c
        x_g = jnp.where(nrs > 0, sub, jnp.maximum(x_g, 0.0))

    emb = lin(jnp.maximum(lin(x_t, w1_ref, b1_ref), 0.0), w2_ref, b2_ref)
    re = lax.broadcasted_iota(jnp.int32, (32, 64), 0)
    ce = lax.broadcasted_iota(jnp.int32, (32, 64), 1)
    sel_e = (ce == 2 * re).astype(f32)
    sel_o = (ce == 2 * re + 1).astype(f32)
    pair = jnp.concatenate([mm(sel_e, emb), mm(sel_o, emb)], axis=1)
    h1 = jnp.maximum(lin(pair, wp1_ref, bp1_ref), 0.0)
    h2 = jnp.maximum(lin(h1, wp2_ref, bp2_ref), 0.0)
    sc = mm(h2, wp3_ref[...]) + bp3_ref[...]
    out_ref[...] = 1.0 / (1.0 + jnp.exp(-sc))


def _tc_dense(args, interpret=False):
    return pl.pallas_call(
        _tc_body,
        out_shape=jax.ShapeDtypeStruct((32, 1), jnp.float32),
        interpret=interpret,
    )(*args)


def kernel(node_features, edge_index, edge_timestamps, target_pairs, params):
    i32 = jnp.int32
    tgt_ids = target_pairs.reshape(-1).astype(i32)

    tgtid, nbrid, tsg, g_rows, t_rows = _sc_compact(
        edge_index.reshape(-1), edge_timestamps, tgt_ids, node_features)

    p = params
    args = [g_rows, t_rows,
            tgtid.reshape(E, 1), nbrid.reshape(E, 1), tsg.reshape(E, 1),
            tgt_ids.reshape(1, 64),
            p['W_in'], p['b_in']]
    for lp in p['layers']:
        args += [lp['Wf'], lp['bf'], lp['Wt1'], lp['bt1'], lp['Wt2'],
                 lp['bt2'], lp['in_w'], lp['in_b'], lp['out_w'], lp['out_b'],
                 lp['Wo'], lp['bo']]
    args += [p['W1'], p['b1'], p['W2'], p['b2'],
             p['Wp1'], p['bp1'], p['Wp2'], p['bp2'], p['Wp3'].T,
             p['bp3'].reshape(1, 1)]
    return _tc_dense(args)


# speedup vs baseline: 1.0129x; 1.0018x over previous
import functools

import jax
import jax.numpy as jnp
import numpy as np
from jax import lax
from jax.experimental import pallas as pl
from jax.experimental.pallas import tpu as pltpu
from jax.experimental.pallas import tpu_sc as plsc

NW = 32
CAP = 128
E = NW * CAP
N_NODES = 10000
N_EDGES = 160000
CHUNK = 5000
TBL = 10248
H = 128
NH = 4
HD = H // NH


def _sc_body(ei_hbm, ts_hbm, tgt_hbm, nf_hbm, zeros_hbm,
             tgtid_out, nbr_out, ts_out, g_out, t_out,
             tbl, e0c, e1c, tsc, tgtv, tgtbuf, nbrbuf, tsbuf, rows, trows,
             sem, sem2):
    wid = lax.axis_index("s") * 2 + lax.axis_index("c")
    base = wid * CHUNK
    c0 = pltpu.async_copy(ei_hbm.at[pl.ds(base, CHUNK)],
                          e0c.at[pl.ds(0, CHUNK)], sem)
    c1 = pltpu.async_copy(ei_hbm.at[pl.ds(N_EDGES + base, CHUNK)],
                          e1c.at[pl.ds(0, CHUNK)], sem)
    c2 = pltpu.async_copy(ts_hbm.at[pl.ds(base, CHUNK)],
                          tsc.at[pl.ds(0, CHUNK)], sem)
    c3 = pltpu.async_copy(tgt_hbm, tgtv, sem)
    c4 = pltpu.async_copy(zeros_hbm, tbl, sem2)

    zeros_f = jnp.zeros((16,), jnp.float32)
    neg_i = jnp.full((16,), -1, jnp.int32)
    ones_i = jnp.ones((16,), jnp.int32)
    lane = lax.iota(jnp.int32, 16)

    for j in range(CAP // 16):
        tgtbuf[pl.ds(j * 16, 16)] = neg_i
        nbrbuf[pl.ds(j * 16, 16)] = lane * 16 + j
        tsbuf[pl.ds(j * 16, 16)] = zeros_f

    c0.wait()
    c1.wait()
    c2.wait()
    c3.wait()
    c4.wait()

    for j in range(64 // 16):
        idx = tgtv[pl.ds(j * 16, 16)]
        plsc.store_scatter(tbl, [idx], ones_i)

    def append16(e0, e1, tv, c):
        f0 = plsc.load_gather(tbl, [e0])
        f1 = plsc.load_gather(tbl, [e1])
        m0 = f0 > 0
        m1 = (f1 > 0) & (e0 != e1)
        anym = jnp.any(m0 | m1)

        def app(c):
            b0 = jnp.minimum(c, CAP - 16)
            plsc.store_compressed(tgtbuf.at[pl.ds(b0, 16)], e0, mask=m0)
            plsc.store_compressed(nbrbuf.at[pl.ds(b0, 16)], e1, mask=m0)
            plsc.store_compressed(tsbuf.at[pl.ds(b0, 16)], tv, mask=m0)
            c = c + jnp.sum(m0.astype(jnp.int32))
            b1 = jnp.minimum(c, CAP - 16)
            plsc.store_compressed(tgtbuf.at[pl.ds(b1, 16)], e1, mask=m1)
            plsc.store_compressed(nbrbuf.at[pl.ds(b1, 16)], e0, mask=m1)
            plsc.store_compressed(tsbuf.at[pl.ds(b1, 16)], tv, mask=m1)
            return c + jnp.sum(m1.astype(jnp.int32))

        return lax.cond(anym, app, lambda c: c, c)

    def body(i, cnt):
        e0 = e0c[pl.ds(i * 16, 16)]
        e1 = e1c[pl.ds(i * 16, 16)]
        tv = tsc[pl.ds(i * 16, 16)]
        return append16(e0, e1, tv, cnt)

    cnt = lax.fori_loop(0, CHUNK // 16, body, jnp.int32(0))

    tail_ok = lane < (CHUNK % 16)
    e0t = jnp.where(tail_ok, e0c[pl.ds(CHUNK - 8, 16)], N_NODES)
    e1t = jnp.where(tail_ok, e1c[pl.ds(CHUNK - 8, 16)], N_NODES)
    tvt = jnp.where(tail_ok, tsc[pl.ds(CHUNK - 8, 16)], 0.0)
    append16(e0t, e1t, tvt, cnt)

    pltpu.async_copy(nf_hbm.at[nbrbuf], rows, sem).wait()

    pltpu.sync_copy(tgtbuf, tgtid_out.at[pl.ds(wid * CAP, CAP)])
    pltpu.sync_copy(nbrbuf, nbr_out.at[pl.ds(wid * CAP, CAP)])
    pltpu.sync_copy(tsbuf, ts_out.at[pl.ds(wid * CAP, CAP)])
    pltpu.sync_copy(rows, g_out.at[pl.ds(wid * CAP, CAP)])

    @pl.when(wid == 0)
    def _():
        pltpu.async_copy(nf_hbm.at[tgtv], trows, sem).wait()
        pltpu.sync_copy(trows, t_out)


def _sc_compact(ei, ts, tgt_ids, node_features, interpret=False):
    f32, i32 = jnp.float32, jnp.int32
    return pl.kernel(
        _sc_body,
        out_type=[
            jax.ShapeDtypeStruct((E,), i32),
            jax.ShapeDtypeStruct((E,), i32),
            jax.ShapeDtypeStruct((E,), f32),
            jax.ShapeDtypeStruct((E, H), f32),
            jax.ShapeDtypeStruct((64, H), f32),
        ],
        mesh=plsc.VectorSubcoreMesh(core_axis_name="c", subcore_axis_name="s"),
        scratch_types=[
            pltpu.VMEM((TBL,), i32),
            pltpu.VMEM((CHUNK + 8,), i32),
            pltpu.VMEM((CHUNK + 8,), i32),
            pltpu.VMEM((CHUNK + 8,), f32),
            pltpu.VMEM((64,), i32),
            pltpu.VMEM((CAP,), i32),
            pltpu.VMEM((CAP,), i32),
            pltpu.VMEM((CAP,), f32),
            pltpu.VMEM((CAP, H), f32),
            pltpu.VMEM((64, H), f32),
            pltpu.SemaphoreType.DMA,
            pltpu.SemaphoreType.DMA,
        ],
        compiler_params=pltpu.CompilerParams(needs_layout_passes=False,
                                             skip_device_barrier=True),
        interpret=interpret,
    )(ei, ts, tgt_ids, node_features, jnp.zeros((TBL,), jnp.int32))


def _tc_body(*refs):
    (g_ref, t_ref, tgtid_ref, nbr_ref, ts_ref, tgtrow_ref,
     w_in_ref, b_in_ref) = refs[:8]
    layer_refs = refs[8:8 + 24]
    (w1_ref, b1_ref, w2_ref, b2_ref, wp1_ref, bp1_ref, wp2_ref, bp2_ref,
     wp3_ref, bp3_ref, out_ref) = refs[8 + 24:]

    f32 = jnp.float32

    def mmt(a, b):
        return lax.dot_general(a, b, (((1,), (1,)), ((), ())),
                               preferred_element_type=f32)

    def mm(a, b):
        return jnp.dot(a, b, preferred_element_type=f32)

    def lin(x, w_ref, b_ref):
        return mmt(x, w_ref[...]) + b_ref[...][None, :]

    tgtid = tgtid_ref[...]
    nbr = nbr_ref[...]
    ts = ts_ref[...]
    tgtrow = tgtrow_ref[...]
    onehot = (tgtid == tgtrow).astype(f32)
    nbrhot = ((nbr == tgtrow) & (tgtid >= 0)).astype(f32)
    validf = (tgtid >= 0).astype(f32)

    r = lax.broadcasted_iota(jnp.int32, (H, NH), 0)
    c = lax.broadcasted_iota(jnp.int32, (H, NH), 1)
    hm = (r // HD == c).astype(f32)
    rt = lax.broadcasted_iota(jnp.int32, (NH, H), 0)
    ct = lax.broadcasted_iota(jnp.int32, (NH, H), 1)
    hmt = (ct // HD == rt).astype(f32)

    x_g = lin(g_ref[...], w_in_ref, b_in_ref)
    x_t = lin(t_ref[...], w_in_ref, b_in_ref)

    rowsum = jnp.maximum(jnp.sum(onehot, axis=1, keepdims=True), 1.0)
    nrs = jnp.sum(nbrhot, axis=1, keepdims=True)
    nrs_c = jnp.maximum(nrs, 1.0)
    inv_sqrt_hd = f32(1.0 / np.sqrt(HD))

    for l in range(2):
        (wf, bf, wt1, bt1, wt2, bt2, in_w, in_b, wout, bout,
         wo, bo) = layer_refs[l * 12:(l + 1) * 12]
        tf_g = lin(x_g, wf, bf)
        tf_t = lin(x_t, wf, bf)
        t1 = jnp.maximum(mmt(ts, wt1[...]) + bt1[...][None, :], 0.0)
        tfeat = lin(t1, wt2, bt2)
        nf = tf_g + tfeat
        in_w_all = in_w[...]
        in_b_all = in_b[...]
        q = mmt(tf_t, in_w_all[:H]) + in_b_all[:H][None, :]
        k = mmt(nf, in_w_all[H:2 * H]) + in_b_all[H:2 * H][None, :]
        v = mmt(nf, in_w_all[2 * H:]) + in_b_all[2 * H:][None, :]
        qrow = mm(onehot, q) / rowsum
        s = mm(qrow * k, hm) * inv_sqrt_hd
        w = jnp.exp(s) * validf
        den = lax.dot_general(onehot, w, (((0,), (0,)), ((), ())),
                              preferred_element_type=f32)
        wv = mm(w, hmt) * v
        num = lax.dot_general(onehot, wv, (((0,), (0,)), ((), ())),
                              preferred_element_type=f32)
        den_rep = mm(den, hmt)
        att = num / jnp.where(den_rep > 0, den_rep, 1.0)
        o = lin(att, wout, bout)
        hasedge = den_rep[:, 0:1] > 0
        agg = jnp.where(hasedge, o, tf_t)
        x_t = jnp.maximum(lin(agg, wo, bo), 0.0)
        sub = mm(nbrhot, x_t) / nrs_c
        x_g = jnp.where(nrs > 0, sub, jnp.maximum(x_g, 0.0))

    emb = lin(jnp.maximum(lin(x_t, w1_ref, b1_ref), 0.0), w2_ref, b2_ref)
    re = lax.broadcasted_iota(jnp.int32, (32, 64), 0)
    ce = lax.broadcasted_iota(jnp.int32, (32, 64), 1)
    sel_e = (ce == 2 * re).astype(f32)
    sel_o = (ce == 2 * re + 1).astype(f32)
    pair = jnp.concatenate([mm(sel_e, emb), mm(sel_o, emb)], axis=1)
    h1 = jnp.maximum(lin(pair, wp1_ref, bp1_ref), 0.0)
    h2 = jnp.maximum(lin(h1, wp2_ref, bp2_ref), 0.0)
    sc = mm(h2, wp3_ref[...]) + bp3_ref[...]
    out_ref[...] = 1.0 / (1.0 + jnp.exp(-sc))


def _tc_dense(args, interpret=False):
    return pl.pallas_call(
        _tc_body,
        out_shape=jax.ShapeDtypeStruct((32, 1), jnp.float32),
        compiler_params=pltpu.CompilerParams(skip_device_barrier=True),
        interpret=interpret,
    )(*args)


def kernel(node_features, edge_index, edge_timestamps, target_pairs, params):
    i32 = jnp.int32
    tgt_ids = target_pairs.reshape(-1).astype(i32)

    tgtid, nbrid, tsg, g_rows, t_rows = _sc_compact(
        edge_index.reshape(-1), edge_timestamps, tgt_ids, node_features)

    p = params
    args = [g_rows, t_rows,
            tgtid.reshape(E, 1), nbrid.reshape(E, 1), tsg.reshape(E, 1),
            tgt_ids.reshape(1, 64),
            p['W_in'], p['b_in']]
    for lp in p['layers']:
        args += [lp['Wf'], lp['bf'], lp['Wt1'], lp['bt1'], lp['Wt2'],
                 lp['bt2'], lp['in_w'], lp['in_b'], lp['out_w'], lp['out_b'],
                 lp['Wo'], lp['bo']]
    args += [p['W1'], p['b1'], p['W2'], p['b2'],
             p['Wp1'], p['bp1'], p['Wp2'], p['bp2'], p['Wp3'].T,
             p['bp3'].reshape(1, 1)]
    return _tc_dense(args)


# speedup vs baseline: 1.0156x; 1.0026x over previous
import functools

import jax
import jax.numpy as jnp
import numpy as np
from jax import lax
from jax.experimental import pallas as pl
from jax.experimental.pallas import tpu as pltpu
from jax.experimental.pallas import tpu_sc as plsc

NW = 32
CAP = 128
E = NW * CAP
N_NODES = 10000
N_EDGES = 160000
CHUNK = 5000
TBL = 10248
H = 128
NH = 4
HD = H // NH


def _sc_body(ei_hbm, ts_hbm, tgt_hbm, nf_hbm, zeros_hbm,
             tgtid_out, nbr_out, ts_out, g_out, t_out,
             tbl, e0c, e1c, tsc, tgtv, tgtbuf, nbrbuf, tsbuf, rows, trows,
             sem, sem2):
    wid = lax.axis_index("s") * 2 + lax.axis_index("c")
    base = wid * CHUNK
    c0 = pltpu.async_copy(ei_hbm.at[pl.ds(base, CHUNK)],
                          e0c.at[pl.ds(0, CHUNK)], sem)
    c1 = pltpu.async_copy(ei_hbm.at[pl.ds(N_EDGES + base, CHUNK)],
                          e1c.at[pl.ds(0, CHUNK)], sem)
    c2 = pltpu.async_copy(ts_hbm.at[pl.ds(base, CHUNK)],
                          tsc.at[pl.ds(0, CHUNK)], sem)
    c3 = pltpu.async_copy(tgt_hbm, tgtv, sem)
    c4 = pltpu.async_copy(zeros_hbm, tbl, sem2)

    zeros_f = jnp.zeros((16,), jnp.float32)
    neg_i = jnp.full((16,), -1, jnp.int32)
    ones_i = jnp.ones((16,), jnp.int32)
    lane = lax.iota(jnp.int32, 16)

    for j in range(CAP // 16):
        tgtbuf[pl.ds(j * 16, 16)] = neg_i
        nbrbuf[pl.ds(j * 16, 16)] = lane * 16 + j
        tsbuf[pl.ds(j * 16, 16)] = zeros_f

    c0.wait()
    c1.wait()
    c2.wait()
    c3.wait()
    c4.wait()

    for j in range(64 // 16):
        idx = tgtv[pl.ds(j * 16, 16)]
        plsc.store_scatter(tbl, [idx], ones_i)

    def append16(e0, e1, tv, c):
        f0 = plsc.load_gather(tbl, [e0])
        f1 = plsc.load_gather(tbl, [e1])
        m0 = f0 > 0
        m1 = (f1 > 0) & (e0 != e1)
        anym = jnp.any(m0 | m1)

        def app(c):
            b0 = jnp.minimum(c, CAP - 16)
            plsc.store_compressed(tgtbuf.at[pl.ds(b0, 16)], e0, mask=m0)
            plsc.store_compressed(nbrbuf.at[pl.ds(b0, 16)], e1, mask=m0)
            plsc.store_compressed(tsbuf.at[pl.ds(b0, 16)], tv, mask=m0)
            c = c + jnp.sum(m0.astype(jnp.int32))
            b1 = jnp.minimum(c, CAP - 16)
            plsc.store_compressed(tgtbuf.at[pl.ds(b1, 16)], e1, mask=m1)
            plsc.store_compressed(nbrbuf.at[pl.ds(b1, 16)], e0, mask=m1)
            plsc.store_compressed(tsbuf.at[pl.ds(b1, 16)], tv, mask=m1)
            return c + jnp.sum(m1.astype(jnp.int32))

        return lax.cond(anym, app, lambda c: c, c)

    def body(i, cnt):
        for u in range(4):
            off = i * 64 + u * 16
            e0 = e0c[pl.ds(off, 16)]
            e1 = e1c[pl.ds(off, 16)]
            tv = tsc[pl.ds(off, 16)]
            cnt = append16(e0, e1, tv, cnt)
        return cnt

    cnt = lax.fori_loop(0, CHUNK // 64, body, jnp.int32(0))
    for u in range(CHUNK // 16 - (CHUNK // 64) * 4):
        off = (CHUNK // 64) * 64 + u * 16
        cnt = append16(e0c[pl.ds(off, 16)], e1c[pl.ds(off, 16)],
                       tsc[pl.ds(off, 16)], cnt)

    tail_ok = lane < (CHUNK % 16)
    e0t = jnp.where(tail_ok, e0c[pl.ds(CHUNK - 8, 16)], N_NODES)
    e1t = jnp.where(tail_ok, e1c[pl.ds(CHUNK - 8, 16)], N_NODES)
    tvt = jnp.where(tail_ok, tsc[pl.ds(CHUNK - 8, 16)], 0.0)
    append16(e0t, e1t, tvt, cnt)

    pltpu.async_copy(nf_hbm.at[nbrbuf], rows, sem).wait()

    pltpu.sync_copy(tgtbuf, tgtid_out.at[pl.ds(wid * CAP, CAP)])
    pltpu.sync_copy(nbrbuf, nbr_out.at[pl.ds(wid * CAP, CAP)])
    pltpu.sync_copy(tsbuf, ts_out.at[pl.ds(wid * CAP, CAP)])
    pltpu.sync_copy(rows, g_out.at[pl.ds(wid * CAP, CAP)])

    @pl.when(wid == 0)
    def _():
        pltpu.async_copy(nf_hbm.at[tgtv], trows, sem).wait()
        pltpu.sync_copy(trows, t_out)


def _sc_compact(ei, ts, tgt_ids, node_features, interpret=False):
    f32, i32 = jnp.float32, jnp.int32
    return pl.kernel(
        _sc_body,
        out_type=[
            jax.ShapeDtypeStruct((E,), i32),
            jax.ShapeDtypeStruct((E,), i32),
            jax.ShapeDtypeStruct((E,), f32),
            jax.ShapeDtypeStruct((E, H), f32),
            jax.ShapeDtypeStruct((64, H), f32),
        ],
        mesh=plsc.VectorSubcoreMesh(core_axis_name="c", subcore_axis_name="s"),
        scratch_types=[
            pltpu.VMEM((TBL,), i32),
            pltpu.VMEM((CHUNK + 8,), i32),
            pltpu.VMEM((CHUNK + 8,), i32),
            pltpu.VMEM((CHUNK + 8,), f32),
            pltpu.VMEM((64,), i32),
            pltpu.VMEM((CAP,), i32),
            pltpu.VMEM((CAP,), i32),
            pltpu.VMEM((CAP,), f32),
            pltpu.VMEM((CAP, H), f32),
            pltpu.VMEM((64, H), f32),
            pltpu.SemaphoreType.DMA,
            pltpu.SemaphoreType.DMA,
        ],
        compiler_params=pltpu.CompilerParams(needs_layout_passes=False,
                                             skip_device_barrier=True),
        interpret=interpret,
    )(ei, ts, tgt_ids, node_features, jnp.zeros((TBL,), jnp.int32))


def _tc_body(*refs):
    (g_ref, t_ref, tgtid_ref, nbr_ref, ts_ref, tgtrow_ref,
     w_in_ref, b_in_ref) = refs[:8]
    layer_refs = refs[8:8 + 24]
    (w1_ref, b1_ref, w2_ref, b2_ref, wp1_ref, bp1_ref, wp2_ref, bp2_ref,
     wp3_ref, bp3_ref, out_ref) = refs[8 + 24:]

    f32 = jnp.float32

    def mmt(a, b):
        return lax.dot_general(a, b, (((1,), (1,)), ((), ())),
                               preferred_element_type=f32)

    def mm(a, b):
        return jnp.dot(a, b, preferred_element_type=f32)

    def lin(x, w_ref, b_ref):
        return mmt(x, w_ref[...]) + b_ref[...][None, :]

    tgtid = tgtid_ref[...]
    nbr = nbr_ref[...]
    ts = ts_ref[...]
    tgtrow = tgtrow_ref[...]
    onehot = (tgtid == tgtrow).astype(f32)
    nbrhot = ((nbr == tgtrow) & (tgtid >= 0)).astype(f32)
    validf = (tgtid >= 0).astype(f32)

    r = lax.broadcasted_iota(jnp.int32, (H, NH), 0)
    c = lax.broadcasted_iota(jnp.int32, (H, NH), 1)
    hm = (r // HD == c).astype(f32)
    rt = lax.broadcasted_iota(jnp.int32, (NH, H), 0)
    ct = lax.broadcasted_iota(jnp.int32, (NH, H), 1)
    hmt = (ct // HD == rt).astype(f32)

    x_g = lin(g_ref[...], w_in_ref, b_in_ref)
    x_t = lin(t_ref[...], w_in_ref, b_in_ref)

    rowsum = jnp.maximum(jnp.sum(onehot, axis=1, keepdims=True), 1.0)
    nrs = jnp.sum(nbrhot, axis=1, keepdims=True)
    nrs_c = jnp.maximum(nrs, 1.0)
    inv_sqrt_hd = f32(1.0 / np.sqrt(HD))

    for l in range(2):
        (wf, bf, wt1, bt1, wt2, bt2, in_w, in_b, wout, bout,
         wo, bo) = layer_refs[l * 12:(l + 1) * 12]
        tf_g = lin(x_g, wf, bf)
        tf_t = lin(x_t, wf, bf)
        t1 = jnp.maximum(mmt(ts, wt1[...]) + bt1[...][None, :], 0.0)
        tfeat = lin(t1, wt2, bt2)
        nf = tf_g + tfeat
        in_w_all = in_w[...]
        in_b_all = in_b[...]
        q = mmt(tf_t, in_w_all[:H]) + in_b_all[:H][None, :]
        k = mmt(nf, in_w_all[H:2 * H]) + in_b_all[H:2 * H][None, :]
        v = mmt(nf, in_w_all[2 * H:]) + in_b_all[2 * H:][None, :]
        qrow = mm(onehot, q) / rowsum
        s = mm(qrow * k, hm) * inv_sqrt_hd
        w = jnp.exp(s) * validf
        den = lax.dot_general(onehot, w, (((0,), (0,)), ((), ())),
                              preferred_element_type=f32)
        wv = mm(w, hmt) * v
        num = lax.dot_general(onehot, wv, (((0,), (0,)), ((), ())),
                              preferred_element_type=f32)
        den_rep = mm(den, hmt)
        att = num / jnp.where(den_rep > 0, den_rep, 1.0)
        o = lin(att, wout, bout)
        hasedge = den_rep[:, 0:1] > 0
        agg = jnp.where(hasedge, o, tf_t)
        x_t = jnp.maximum(lin(agg, wo, bo), 0.0)
        sub = mm(nbrhot, x_t) / nrs_c
        x_g = jnp.where(nrs > 0, sub, jnp.maximum(x_g, 0.0))

    emb = lin(jnp.maximum(lin(x_t, w1_ref, b1_ref), 0.0), w2_ref, b2_ref)
    re = lax.broadcasted_iota(jnp.int32, (32, 64), 0)
    ce = lax.broadcasted_iota(jnp.int32, (32, 64), 1)
    sel_e = (ce == 2 * re).astype(f32)
    sel_o = (ce == 2 * re + 1).astype(f32)
    pair = jnp.concatenate([mm(sel_e, emb), mm(sel_o, emb)], axis=1)
    h1 = jnp.maximum(lin(pair, wp1_ref, bp1_ref), 0.0)
    h2 = jnp.maximum(lin(h1, wp2_ref, bp2_ref), 0.0)
    sc = mm(h2, wp3_ref[...]) + bp3_ref[...]
    out_ref[...] = 1.0 / (1.0 + jnp.exp(-sc))


def _tc_dense(args, interpret=False):
    return pl.pallas_call(
        _tc_body,
        out_shape=jax.ShapeDtypeStruct((32, 1), jnp.float32),
        compiler_params=pltpu.CompilerParams(skip_device_barrier=True),
        interpret=interpret,
    )(*args)


def kernel(node_features, edge_index, edge_timestamps, target_pairs, params):
    i32 = jnp.int32
    tgt_ids = target_pairs.reshape(-1).astype(i32)

    tgtid, nbrid, tsg, g_rows, t_rows = _sc_compact(
        edge_index.reshape(-1), edge_timestamps, tgt_ids, node_features)

    p = params
    args = [g_rows, t_rows,
            tgtid.reshape(E, 1), nbrid.reshape(E, 1), tsg.reshape(E, 1),
            tgt_ids.reshape(1, 64),
            p['W_in'], p['b_in']]
    for lp in p['layers']:
        args += [lp['Wf'], lp['bf'], lp['Wt1'], lp['bt1'], lp['Wt2'],
                 lp['bt2'], lp['in_w'], lp['in_b'], lp['out_w'], lp['out_b'],
                 lp['Wo'], lp['bo']]
    args += [p['W1'], p['b1'], p['W2'], p['b2'],
             p['Wp1'], p['bp1'], p['Wp2'], p['bp2'], p['Wp3'].T,
             p['bp3'].reshape(1, 1)]
    return _tc_dense(args)


# speedup vs baseline: 1.1035x; 1.0865x over previous
import functools

import jax
import jax.numpy as jnp
import numpy as np
from jax import lax
from jax.experimental import pallas as pl
from jax.experimental.pallas import tpu as pltpu
from jax.experimental.pallas import tpu_sc as plsc

NW = 32
CAP = 128
E = NW * CAP
N_NODES = 10000
N_EDGES = 160000
CHUNK = 5000
TBL = 10248
H = 128
NH = 4
HD = H // NH


def _sc_body(ei_hbm, ts_hbm, tgt_hbm, nf_hbm, zeros_hbm,
             tgtid_out, nbr_out, ts_out, g_out, t_out,
             tbl, e0c, e1c, tsc, tgtv, tgtbuf, nbrbuf, tsbuf, rows, trows,
             sem, sem2):
    wid = lax.axis_index("s") * 2 + lax.axis_index("c")
    base = wid * CHUNK
    c0 = pltpu.async_copy(ei_hbm.at[pl.ds(base, CHUNK)],
                          e0c.at[pl.ds(0, CHUNK)], sem)
    c1 = pltpu.async_copy(ei_hbm.at[pl.ds(N_EDGES + base, CHUNK)],
                          e1c.at[pl.ds(0, CHUNK)], sem)
    c2 = pltpu.async_copy(ts_hbm.at[pl.ds(base, CHUNK)],
                          tsc.at[pl.ds(0, CHUNK)], sem)
    c3 = pltpu.async_copy(tgt_hbm, tgtv, sem)
    c4 = pltpu.async_copy(zeros_hbm, tbl, sem2)

    zeros_f = jnp.zeros((16,), jnp.float32)
    neg_i = jnp.full((16,), -1, jnp.int32)
    ones_i = jnp.ones((16,), jnp.int32)
    lane = lax.iota(jnp.int32, 16)

    for j in range(CAP // 16):
        tgtbuf[pl.ds(j * 16, 16)] = neg_i
        nbrbuf[pl.ds(j * 16, 16)] = lane * 16 + j
        tsbuf[pl.ds(j * 16, 16)] = zeros_f

    c0.wait()
    c1.wait()
    c2.wait()
    c3.wait()
    c4.wait()

    for j in range(64 // 16):
        idx = tgtv[pl.ds(j * 16, 16)]
        plsc.store_scatter(tbl, [idx], ones_i)

    def append16(e0, e1, tv, c):
        f0 = plsc.load_gather(tbl, [e0])
        f1 = plsc.load_gather(tbl, [e1])
        m0 = f0 > 0
        m1 = (f1 > 0) & (e0 != e1)
        b0 = jnp.minimum(c, CAP - 16)
        plsc.store_compressed(tgtbuf.at[pl.ds(b0, 16)], e0, mask=m0)
        plsc.store_compressed(nbrbuf.at[pl.ds(b0, 16)], e1, mask=m0)
        plsc.store_compressed(tsbuf.at[pl.ds(b0, 16)], tv, mask=m0)
        c = c + plsc.all_reduce_population_count(m0)[0]
        b1 = jnp.minimum(c, CAP - 16)
        plsc.store_compressed(tgtbuf.at[pl.ds(b1, 16)], e1, mask=m1)
        plsc.store_compressed(nbrbuf.at[pl.ds(b1, 16)], e0, mask=m1)
        plsc.store_compressed(tsbuf.at[pl.ds(b1, 16)], tv, mask=m1)
        return c + plsc.all_reduce_population_count(m1)[0]

    def body(i, cnt):
        e0 = e0c[pl.ds(i * 16, 16)]
        e1 = e1c[pl.ds(i * 16, 16)]
        tv = tsc[pl.ds(i * 16, 16)]
        return append16(e0, e1, tv, cnt)

    cnt = lax.fori_loop(0, CHUNK // 16, body, jnp.int32(0))

    tail_ok = lane < (CHUNK % 16)
    e0t = jnp.where(tail_ok, e0c[pl.ds(CHUNK - 8, 16)], N_NODES)
    e1t = jnp.where(tail_ok, e1c[pl.ds(CHUNK - 8, 16)], N_NODES)
    tvt = jnp.where(tail_ok, tsc[pl.ds(CHUNK - 8, 16)], 0.0)
    append16(e0t, e1t, tvt, cnt)

    pltpu.async_copy(nf_hbm.at[nbrbuf], rows, sem).wait()

    pltpu.sync_copy(tgtbuf, tgtid_out.at[pl.ds(wid * CAP, CAP)])
    pltpu.sync_copy(nbrbuf, nbr_out.at[pl.ds(wid * CAP, CAP)])
    pltpu.sync_copy(tsbuf, ts_out.at[pl.ds(wid * CAP, CAP)])
    pltpu.sync_copy(rows, g_out.at[pl.ds(wid * CAP, CAP)])

    @pl.when(wid == 0)
    def _():
        pltpu.async_copy(nf_hbm.at[tgtv], trows, sem).wait()
        pltpu.sync_copy(trows, t_out)


def _sc_compact(ei, ts, tgt_ids, node_features, interpret=False):
    f32, i32 = jnp.float32, jnp.int32
    return pl.kernel(
        _sc_body,
        out_type=[
            jax.ShapeDtypeStruct((E,), i32),
            jax.ShapeDtypeStruct((E,), i32),
            jax.ShapeDtypeStruct((E,), f32),
            jax.ShapeDtypeStruct((E, H), f32),
            jax.ShapeDtypeStruct((64, H), f32),
        ],
        mesh=plsc.VectorSubcoreMesh(core_axis_name="c", subcore_axis_name="s"),
        scratch_types=[
            pltpu.VMEM((TBL,), i32),
            pltpu.VMEM((CHUNK + 8,), i32),
            pltpu.VMEM((CHUNK + 8,), i32),
            pltpu.VMEM((CHUNK + 8,), f32),
            pltpu.VMEM((64,), i32),
            pltpu.VMEM((CAP,), i32),
            pltpu.VMEM((CAP,), i32),
            pltpu.VMEM((CAP,), f32),
            pltpu.VMEM((CAP, H), f32),
            pltpu.VMEM((64, H), f32),
            pltpu.SemaphoreType.DMA,
            pltpu.SemaphoreType.DMA,
        ],
        compiler_params=pltpu.CompilerParams(needs_layout_passes=False,
                                             skip_device_barrier=True),
        interpret=interpret,
    )(ei, ts, tgt_ids, node_features, jnp.zeros((TBL,), jnp.int32))


def _tc_body(*refs):
    (g_ref, t_ref, tgtid_ref, nbr_ref, ts_ref, tgtrow_ref,
     w_in_ref, b_in_ref) = refs[:8]
    layer_refs = refs[8:8 + 24]
    (w1_ref, b1_ref, w2_ref, b2_ref, wp1_ref, bp1_ref, wp2_ref, bp2_ref,
     wp3_ref, bp3_ref, out_ref) = refs[8 + 24:]

    f32 = jnp.float32

    def mmt(a, b):
        return lax.dot_general(a, b, (((1,), (1,)), ((), ())),
                               preferred_element_type=f32)

    def mm(a, b):
        return jnp.dot(a, b, preferred_element_type=f32)

    def lin(x, w_ref, b_ref):
        return mmt(x, w_ref[...]) + b_ref[...][None, :]

    tgtid = tgtid_ref[...]
    nbr = nbr_ref[...]
    ts = ts_ref[...]
    tgtrow = tgtrow_ref[...]
    onehot = (tgtid == tgtrow).astype(f32)
    nbrhot = ((nbr == tgtrow) & (tgtid >= 0)).astype(f32)
    validf = (tgtid >= 0).astype(f32)

    r = lax.broadcasted_iota(jnp.int32, (H, NH), 0)
    c = lax.broadcasted_iota(jnp.int32, (H, NH), 1)
    hm = (r // HD == c).astype(f32)
    rt = lax.broadcasted_iota(jnp.int32, (NH, H), 0)
    ct = lax.broadcasted_iota(jnp.int32, (NH, H), 1)
    hmt = (ct // HD == rt).astype(f32)

    x_g = lin(g_ref[...], w_in_ref, b_in_ref)
    x_t = lin(t_ref[...], w_in_ref, b_in_ref)

    rowsum = jnp.maximum(jnp.sum(onehot, axis=1, keepdims=True), 1.0)
    nrs = jnp.sum(nbrhot, axis=1, keepdims=True)
    nrs_c = jnp.maximum(nrs, 1.0)
    inv_sqrt_hd = f32(1.0 / np.sqrt(HD))

    for l in range(2):
        (wf, bf, wt1, bt1, wt2, bt2, in_w, in_b, wout, bout,
         wo, bo) = layer_refs[l * 12:(l + 1) * 12]
        tf_g = lin(x_g, wf, bf)
        tf_t = lin(x_t, wf, bf)
        t1 = jnp.maximum(mmt(ts, wt1[...]) + bt1[...][None, :], 0.0)
        tfeat = lin(t1, wt2, bt2)
        nf = tf_g + tfeat
        in_w_all = in_w[...]
        in_b_all = in_b[...]
        q = mmt(tf_t, in_w_all[:H]) + in_b_all[:H][None, :]
        k = mmt(nf, in_w_all[H:2 * H]) + in_b_all[H:2 * H][None, :]
        v = mmt(nf, in_w_all[2 * H:]) + in_b_all[2 * H:][None, :]
        qrow = mm(onehot, q) / rowsum
        s = mm(qrow * k, hm) * inv_sqrt_hd
        w = jnp.exp(s) * validf
        den = lax.dot_general(onehot, w, (((0,), (0,)), ((), ())),
                              preferred_element_type=f32)
        wv = mm(w, hmt) * v
        num = lax.dot_general(onehot, wv, (((0,), (0,)), ((), ())),
                              preferred_element_type=f32)
        den_rep = mm(den, hmt)
        att = num / jnp.where(den_rep > 0, den_rep, 1.0)
        o = lin(att, wout, bout)
        hasedge = den_rep[:, 0:1] > 0
        agg = jnp.where(hasedge, o, tf_t)
        x_t = jnp.maximum(lin(agg, wo, bo), 0.0)
        sub = mm(nbrhot, x_t) / nrs_c
        x_g = jnp.where(nrs > 0, sub, jnp.maximum(x_g, 0.0))

    emb = lin(jnp.maximum(lin(x_t, w1_ref, b1_ref), 0.0), w2_ref, b2_ref)
    re = lax.broadcasted_iota(jnp.int32, (32, 64), 0)
    ce = lax.broadcasted_iota(jnp.int32, (32, 64), 1)
    sel_e = (ce == 2 * re).astype(f32)
    sel_o = (ce == 2 * re + 1).astype(f32)
    pair = jnp.concatenate([mm(sel_e, emb), mm(sel_o, emb)], axis=1)
    h1 = jnp.maximum(lin(pair, wp1_ref, bp1_ref), 0.0)
    h2 = jnp.maximum(lin(h1, wp2_ref, bp2_ref), 0.0)
    sc = mm(h2, wp3_ref[...]) + bp3_ref[...]
    out_ref[...] = 1.0 / (1.0 + jnp.exp(-sc))


def _tc_dense(args, interpret=False):
    return pl.pallas_call(
        _tc_body,
        out_shape=jax.ShapeDtypeStruct((32, 1), jnp.float32),
        compiler_params=pltpu.CompilerParams(skip_device_barrier=True),
        interpret=interpret,
    )(*args)


def kernel(node_features, edge_index, edge_timestamps, target_pairs, params):
    i32 = jnp.int32
    tgt_ids = target_pairs.reshape(-1).astype(i32)

    tgtid, nbrid, tsg, g_rows, t_rows = _sc_compact(
        edge_index.reshape(-1), edge_timestamps, tgt_ids, node_features)

    p = params
    args = [g_rows, t_rows,
            tgtid.reshape(E, 1), nbrid.reshape(E, 1), tsg.reshape(E, 1),
            tgt_ids.reshape(1, 64),
            p['W_in'], p['b_in']]
    for lp in p['layers']:
        args += [lp['Wf'], lp['bf'], lp['Wt1'], lp['bt1'], lp['Wt2'],
                 lp['bt2'], lp['in_w'], lp['in_b'], lp['out_w'], lp['out_b'],
                 lp['Wo'], lp['bo']]
    args += [p['W1'], p['b1'], p['W2'], p['b2'],
             p['Wp1'], p['bp1'], p['Wp2'], p['bp2'], p['Wp3'].T,
             p['bp3'].reshape(1, 1)]
    return _tc_dense(args)


# speedup vs baseline: 1.1050x; 1.0014x over previous
import functools

import jax
import jax.numpy as jnp
import numpy as np
from jax import lax
from jax.experimental import pallas as pl
from jax.experimental.pallas import tpu as pltpu
from jax.experimental.pallas import tpu_sc as plsc

NW = 32
CAP = 128
E = NW * CAP
N_NODES = 10000
N_EDGES = 160000
CHUNK = 5000
TBL = 10248
H = 128
NH = 4
HD = H // NH


def _sc_body(ei_hbm, ts_hbm, tgt_hbm, nf_hbm, zeros_hbm,
             tgtid_out, nbr_out, ts_out, g_out, t_out,
             tbl, e0c, e1c, tsc, tgtv, tgtbuf, nbrbuf, tsbuf, rows, trows,
             sem, sem2):
    wid = lax.axis_index("s") * 2 + lax.axis_index("c")
    base = wid * CHUNK
    c0 = pltpu.async_copy(ei_hbm.at[pl.ds(base, CHUNK)],
                          e0c.at[pl.ds(0, CHUNK)], sem)
    c1 = pltpu.async_copy(ei_hbm.at[pl.ds(N_EDGES + base, CHUNK)],
                          e1c.at[pl.ds(0, CHUNK)], sem)
    c2 = pltpu.async_copy(ts_hbm.at[pl.ds(base, CHUNK)],
                          tsc.at[pl.ds(0, CHUNK)], sem)
    c3 = pltpu.async_copy(tgt_hbm, tgtv, sem)
    c4 = pltpu.async_copy(zeros_hbm, tbl, sem2)

    zeros_f = jnp.zeros((16,), jnp.float32)
    neg_i = jnp.full((16,), -1, jnp.int32)
    ones_i = jnp.ones((16,), jnp.int32)
    lane = lax.iota(jnp.int32, 16)

    for j in range(CAP // 16):
        tgtbuf[pl.ds(j * 16, 16)] = neg_i
        nbrbuf[pl.ds(j * 16, 16)] = lane * 16 + j
        tsbuf[pl.ds(j * 16, 16)] = zeros_f

    c0.wait()
    c1.wait()
    c2.wait()
    c3.wait()
    c4.wait()

    for j in range(64 // 16):
        idx = tgtv[pl.ds(j * 16, 16)]
        plsc.store_scatter(tbl, [idx], ones_i)

    def append16(e0, e1, tv, c):
        f0 = plsc.load_gather(tbl, [e0])
        f1 = plsc.load_gather(tbl, [e1])
        m0 = f0 > 0
        m1 = (f1 > 0) & (e0 != e1)
        b0 = jnp.minimum(c, CAP - 16)
        plsc.store_compressed(tgtbuf.at[pl.ds(b0, 16)], e0, mask=m0)
        plsc.store_compressed(nbrbuf.at[pl.ds(b0, 16)], e1, mask=m0)
        plsc.store_compressed(tsbuf.at[pl.ds(b0, 16)], tv, mask=m0)
        c = c + plsc.all_reduce_population_count(m0)[0]
        b1 = jnp.minimum(c, CAP - 16)
        plsc.store_compressed(tgtbuf.at[pl.ds(b1, 16)], e1, mask=m1)
        plsc.store_compressed(nbrbuf.at[pl.ds(b1, 16)], e0, mask=m1)
        plsc.store_compressed(tsbuf.at[pl.ds(b1, 16)], tv, mask=m1)
        return c + plsc.all_reduce_population_count(m1)[0]

    def body(i, cnt):
        e0 = e0c[pl.ds(i * 16, 16)]
        e1 = e1c[pl.ds(i * 16, 16)]
        tv = tsc[pl.ds(i * 16, 16)]
        return append16(e0, e1, tv, cnt)

    cnt = lax.fori_loop(0, CHUNK // 16, body, jnp.int32(0))

    tail_ok = lane < (CHUNK % 16)
    e0t = jnp.where(tail_ok, e0c[pl.ds(CHUNK - 8, 16)], N_NODES)
    e1t = jnp.where(tail_ok, e1c[pl.ds(CHUNK - 8, 16)], N_NODES)
    tvt = jnp.where(tail_ok, tsc[pl.ds(CHUNK - 8, 16)], 0.0)
    append16(e0t, e1t, tvt, cnt)

    gat = pltpu.async_copy(nf_hbm.at[nbrbuf], rows, sem)
    o0 = pltpu.async_copy(tgtbuf, tgtid_out.at[pl.ds(wid * CAP, CAP)], sem2)
    o1 = pltpu.async_copy(nbrbuf, nbr_out.at[pl.ds(wid * CAP, CAP)], sem2)
    o2 = pltpu.async_copy(tsbuf, ts_out.at[pl.ds(wid * CAP, CAP)], sem2)
    gat.wait()
    pltpu.sync_copy(rows, g_out.at[pl.ds(wid * CAP, CAP)])
    o0.wait()
    o1.wait()
    o2.wait()

    @pl.when(wid == 0)
    def _():
        pltpu.async_copy(nf_hbm.at[tgtv], trows, sem).wait()
        pltpu.sync_copy(trows, t_out)


def _sc_compact(ei, ts, tgt_ids, node_features, interpret=False):
    f32, i32 = jnp.float32, jnp.int32
    return pl.kernel(
        _sc_body,
        out_type=[
            jax.ShapeDtypeStruct((E,), i32),
            jax.ShapeDtypeStruct((E,), i32),
            jax.ShapeDtypeStruct((E,), f32),
            jax.ShapeDtypeStruct((E, H), f32),
            jax.ShapeDtypeStruct((64, H), f32),
        ],
        mesh=plsc.VectorSubcoreMesh(core_axis_name="c", subcore_axis_name="s"),
        scratch_types=[
            pltpu.VMEM((TBL,), i32),
            pltpu.VMEM((CHUNK + 8,), i32),
            pltpu.VMEM((CHUNK + 8,), i32),
            pltpu.VMEM((CHUNK + 8,), f32),
            pltpu.VMEM((64,), i32),
            pltpu.VMEM((CAP,), i32),
            pltpu.VMEM((CAP,), i32),
            pltpu.VMEM((CAP,), f32),
            pltpu.VMEM((CAP, H), f32),
            pltpu.VMEM((64, H), f32),
            pltpu.SemaphoreType.DMA,
            pltpu.SemaphoreType.DMA,
        ],
        compiler_params=pltpu.CompilerParams(needs_layout_passes=False,
                                             skip_device_barrier=True),
        interpret=interpret,
    )(ei, ts, tgt_ids, node_features, jnp.zeros((TBL,), jnp.int32))


def _tc_body(*refs):
    (g_ref, t_ref, tgtid_ref, nbr_ref, ts_ref, tgtrow_ref,
     w_in_ref, b_in_ref) = refs[:8]
    layer_refs = refs[8:8 + 24]
    (w1_ref, b1_ref, w2_ref, b2_ref, wp1_ref, bp1_ref, wp2_ref, bp2_ref,
     wp3_ref, bp3_ref, out_ref) = refs[8 + 24:]

    f32 = jnp.float32

    def mmt(a, b):
        return lax.dot_general(a, b, (((1,), (1,)), ((), ())),
                               preferred_element_type=f32)

    def mm(a, b):
        return jnp.dot(a, b, preferred_element_type=f32)

    def lin(x, w_ref, b_ref):
        return mmt(x, w_ref[...]) + b_ref[...][None, :]

    tgtid = tgtid_ref[...]
    nbr = nbr_ref[...]
    ts = ts_ref[...]
    tgtrow = tgtrow_ref[...]
    onehot = (tgtid == tgtrow).astype(f32)
    nbrhot = ((nbr == tgtrow) & (tgtid >= 0)).astype(f32)
    validf = (tgtid >= 0).astype(f32)

    r = lax.broadcasted_iota(jnp.int32, (H, NH), 0)
    c = lax.broadcasted_iota(jnp.int32, (H, NH), 1)
    hm = (r // HD == c).astype(f32)
    rt = lax.broadcasted_iota(jnp.int32, (NH, H), 0)
    ct = lax.broadcasted_iota(jnp.int32, (NH, H), 1)
    hmt = (ct // HD == rt).astype(f32)

    x_g = lin(g_ref[...], w_in_ref, b_in_ref)
    x_t = lin(t_ref[...], w_in_ref, b_in_ref)

    rowsum = jnp.maximum(jnp.sum(onehot, axis=1, keepdims=True), 1.0)
    nrs = jnp.sum(nbrhot, axis=1, keepdims=True)
    nrs_c = jnp.maximum(nrs, 1.0)
    inv_sqrt_hd = f32(1.0 / np.sqrt(HD))

    for l in range(2):
        (wf, bf, wt1, bt1, wt2, bt2, in_w, in_b, wout, bout,
         wo, bo) = layer_refs[l * 12:(l + 1) * 12]
        tf_g = lin(x_g, wf, bf)
        tf_t = lin(x_t, wf, bf)
        t1 = jnp.maximum(mmt(ts, wt1[...]) + bt1[...][None, :], 0.0)
        tfeat = lin(t1, wt2, bt2)
        nf = tf_g + tfeat
        in_w_all = in_w[...]
        in_b_all = in_b[...]
        q = mmt(tf_t, in_w_all[:H]) + in_b_all[:H][None, :]
        k = mmt(nf, in_w_all[H:2 * H]) + in_b_all[H:2 * H][None, :]
        v = mmt(nf, in_w_all[2 * H:]) + in_b_all[2 * H:][None, :]
        qrow = mm(onehot, q) / rowsum
        s = mm(qrow * k, hm) * inv_sqrt_hd
        w = jnp.exp(s) * validf
        den = lax.dot_general(onehot, w, (((0,), (0,)), ((), ())),
                              preferred_element_type=f32)
        wv = mm(w, hmt) * v
        num = lax.dot_general(onehot, wv, (((0,), (0,)), ((), ())),
                              preferred_element_type=f32)
        den_rep = mm(den, hmt)
        att = num / jnp.where(den_rep > 0, den_rep, 1.0)
        o = lin(att, wout, bout)
        hasedge = den_rep[:, 0:1] > 0
        agg = jnp.where(hasedge, o, tf_t)
        x_t = jnp.maximum(lin(agg, wo, bo), 0.0)
        sub = mm(nbrhot, x_t) / nrs_c
        x_g = jnp.where(nrs > 0, sub, jnp.maximum(x_g, 0.0))

    emb = lin(jnp.maximum(lin(x_t, w1_ref, b1_ref), 0.0), w2_ref, b2_ref)
    re = lax.broadcasted_iota(jnp.int32, (32, 64), 0)
    ce = lax.broadcasted_iota(jnp.int32, (32, 64), 1)
    sel_e = (ce == 2 * re).astype(f32)
    sel_o = (ce == 2 * re + 1).astype(f32)
    pair = jnp.concatenate([mm(sel_e, emb), mm(sel_o, emb)], axis=1)
    h1 = jnp.maximum(lin(pair, wp1_ref, bp1_ref), 0.0)
    h2 = jnp.maximum(lin(h1, wp2_ref, bp2_ref), 0.0)
    sc = mm(h2, wp3_ref[...]) + bp3_ref[...]
    out_ref[...] = 1.0 / (1.0 + jnp.exp(-sc))


def _tc_dense(args, interpret=False):
    return pl.pallas_call(
        _tc_body,
        out_shape=jax.ShapeDtypeStruct((32, 1), jnp.float32),
        compiler_params=pltpu.CompilerParams(skip_device_barrier=True),
        interpret=interpret,
    )(*args)


def kernel(node_features, edge_index, edge_timestamps, target_pairs, params):
    i32 = jnp.int32
    tgt_ids = target_pairs.reshape(-1).astype(i32)

    tgtid, nbrid, tsg, g_rows, t_rows = _sc_compact(
        edge_index.reshape(-1), edge_timestamps, tgt_ids, node_features)

    p = params
    args = [g_rows, t_rows,
            tgtid.reshape(E, 1), nbrid.reshape(E, 1), tsg.reshape(E, 1),
            tgt_ids.reshape(1, 64),
            p['W_in'], p['b_in']]
    for lp in p['layers']:
        args += [lp['Wf'], lp['bf'], lp['Wt1'], lp['bt1'], lp['Wt2'],
                 lp['bt2'], lp['in_w'], lp['in_b'], lp['out_w'], lp['out_b'],
                 lp['Wo'], lp['bo']]
    args += [p['W1'], p['b1'], p['W2'], p['b2'],
             p['Wp1'], p['bp1'], p['Wp2'], p['bp2'], p['Wp3'].T,
             p['bp3'].reshape(1, 1)]
    return _tc_dense(args)
